# baseline (speedup 1.0000x reference)
import math

import jax
import jax.numpy as jnp
from jax import lax
from jax.experimental import pallas as pl
from jax.experimental.pallas import tpu as pltpu
from jax.experimental.pallas import tpu_sc as plsc

_N = 1000
_NP = 1024
_K = 80
_SCORE_THRESH = 0.05
_NMS_THRESH = 0.5
_DETS = 100
_SCALE_CLAMP = math.log(1000.0 / 16.0)
_NEG = -1e9
_OFFSET = 4096.0

_NC = 2
_NS = 16
_NW = _NC * _NS
_FLAT = _NP * _K
_CHUNK = _FLAT // _NW
_UNROLL = 4
_VECS = _CHUNK // (16 * _UNROLL)
_CAP = 128
_TOT = _NW * _CAP
_ROWS = _TOT // 128


def _dense_body(prop_ref, logits_ref, dx_ref, dy_ref, dw_ref, dh_ref, out_ref):
    logits = logits_ref[...]
    m = jnp.max(logits, axis=1, keepdims=True)
    e = jnp.exp(logits - m)
    probs = e / jnp.sum(e, axis=1, keepdims=True)
    sc = probs[:, :_K]
    out_ref[0, 0:_N, :] = jnp.where(sc > _SCORE_THRESH, sc, _NEG)
    out_ref[0, _N:_NP, :] = jnp.full((_NP - _N, _K), _NEG, jnp.float32)

    p = prop_ref[...]
    w = p[:, 2:3] - p[:, 0:1]
    h = p[:, 3:4] - p[:, 1:2]
    cx = p[:, 0:1] + 0.5 * w
    cy = p[:, 1:2] + 0.5 * h
    dx = dx_ref[...] / 10.0
    dy = dy_ref[...] / 10.0
    dw = jnp.minimum(dw_ref[...] / 5.0, _SCALE_CLAMP)
    dh = jnp.minimum(dh_ref[...] / 5.0, _SCALE_CLAMP)
    pcx = dx * w + cx
    pcy = dy * h + cy
    pw = jnp.exp(dw) * w
    ph = jnp.exp(dh) * h
    off = lax.broadcasted_iota(jnp.int32, (_N, _K), 1).astype(jnp.float32) * _OFFSET
    zpad = jnp.zeros((_NP - _N, _K), jnp.float32)
    out_ref[1, 0:_N, :] = (pcx - 0.5 * pw) + off
    out_ref[1, _N:_NP, :] = zpad
    out_ref[2, 0:_N, :] = (pcy - 0.5 * ph) + off
    out_ref[2, _N:_NP, :] = zpad
    out_ref[3, 0:_N, :] = (pcx + 0.5 * pw) + off
    out_ref[3, _N:_NP, :] = zpad
    out_ref[4, 0:_N, :] = (pcy + 0.5 * ph) + off
    out_ref[4, _N:_NP, :] = zpad


def _dense(proposals, class_logits, dx, dy, dw, dh):
    return pl.pallas_call(
        _dense_body,
        out_shape=jax.ShapeDtypeStruct((5, _NP, _K), jnp.float32),
    )(proposals, class_logits, dx, dy, dw, dh)


def _compact_body(planes_hbm, out_hbm,
                  s_v, x1_v, y1_v, x2_v, y2_v, idx_v,
                  os_v, ox1_v, oy1_v, ox2_v, oy2_v, oid_v):
    cid = lax.axis_index("c")
    sid = lax.axis_index("s")
    wid = sid * _NC + cid
    base = wid * _CHUNK

    pltpu.sync_copy(planes_hbm.at[pl.ds(0 * _FLAT + base, _CHUNK)], s_v)
    pltpu.sync_copy(planes_hbm.at[pl.ds(1 * _FLAT + base, _CHUNK)], x1_v)
    pltpu.sync_copy(planes_hbm.at[pl.ds(2 * _FLAT + base, _CHUNK)], y1_v)
    pltpu.sync_copy(planes_hbm.at[pl.ds(3 * _FLAT + base, _CHUNK)], x2_v)
    pltpu.sync_copy(planes_hbm.at[pl.ds(4 * _FLAT + base, _CHUNK)], y2_v)

    zero16 = jnp.zeros((16,), jnp.int32)
    for j in range(_CAP // 16):
        idx_v[pl.ds(j * 16, 16)] = zero16

    lane = lax.iota(jnp.int32, 16)
    start = jnp.zeros((16,), jnp.int32) + jnp.where(wid == 0, 1, 0)

    def scan_body(v, cnt):
        b = v * (16 * _UNROLL)
        for u in range(_UNROLL):
            sv = s_v[pl.ds(b + u * 16, 16)]
            m = sv > _SCORE_THRESH
            pos = plsc.cumsum(jnp.where(m, 1, 0))
            dst = cnt + pos - 1
            m2 = jnp.logical_and(m, dst < _CAP)
            plsc.store_scatter(idx_v, [dst], lane + (b + u * 16), mask=m2)
            cnt = cnt + plsc.all_reduce_population_count(m2)
        return cnt

    cnt = lax.fori_loop(0, _VECS, scan_body, start)

    w0 = wid == 0
    for j in range(_CAP // 16):
        iv = idx_v[pl.ds(j * 16, 16)]
        gx1 = plsc.load_gather(x1_v, [iv])
        gy1 = plsc.load_gather(y1_v, [iv])
        gx2 = plsc.load_gather(x2_v, [iv])
        gy2 = plsc.load_gather(y2_v, [iv])
        gs = plsc.load_gather(s_v, [iv])
        valid = (lane + j * 16) < cnt
        gs = jnp.where(valid, gs, _NEG)
        if j == 0:
            gs = jnp.where(jnp.logical_and(w0, lane == 0), _NEG, gs)
        sl = pl.ds(j * 16, 16)
        os_v[sl] = gs
        ox1_v[sl] = gx1
        oy1_v[sl] = gy1
        ox2_v[sl] = gx2
        oy2_v[sl] = gy2
        oid_v[sl] = (iv + base).astype(jnp.float32)

    pltpu.sync_copy(os_v, out_hbm.at[0, wid])
    pltpu.sync_copy(ox1_v, out_hbm.at[1, wid])
    pltpu.sync_copy(oy1_v, out_hbm.at[2, wid])
    pltpu.sync_copy(ox2_v, out_hbm.at[3, wid])
    pltpu.sync_copy(oy2_v, out_hbm.at[4, wid])
    pltpu.sync_copy(oid_v, out_hbm.at[5, wid])


def _compact(planes_flat):
    chunk = pltpu.VMEM((_CHUNK,), jnp.float32)
    seg_f = pltpu.VMEM((_CAP,), jnp.float32)
    seg_i = pltpu.VMEM((_CAP,), jnp.int32)
    mesh = plsc.VectorSubcoreMesh(
        core_axis_name="c", subcore_axis_name="s",
        num_cores=_NC, num_subcores=_NS)
    run = pl.kernel(
        _compact_body,
        out_type=jax.ShapeDtypeStruct((6, _NW, _CAP), jnp.float32),
        mesh=mesh,
        scratch_types=[chunk] * 5 + [seg_i, seg_f, seg_f, seg_f, seg_f, seg_f, seg_f],
        compiler_params=pltpu.CompilerParams(needs_layout_passes=False),
    )
    return run(planes_flat)


def _bmax(x):
    r = jnp.max(x, axis=0, keepdims=True)
    for k in (1, 2, 4, 8, 16, 32, 64):
        r = jnp.maximum(r, pltpu.roll(r, k, 1))
    return r


def _bsum(x):
    r = jnp.sum(x, axis=0, keepdims=True)
    for k in (1, 2, 4, 8, 16, 32, 64):
        r = r + pltpu.roll(r, k, 1)
    return r


def _nms_body(tab_ref, out_ref):
    s0v = tab_ref[0]
    fx1 = tab_ref[1]
    fy1 = tab_ref[2]
    fx2 = tab_ref[3]
    fy2 = tab_ref[4]
    fid = tab_ref[5]
    area = (fx2 - fx1) * (fy2 - fy1)
    pos = (lax.broadcasted_iota(jnp.int32, (_ROWS, 128), 0) * 128
           + lax.broadcasted_iota(jnp.int32, (_ROWS, 128), 1))
    lane = lax.broadcasted_iota(jnp.int32, (1, 128), 1)
    s0 = pos == 0
    sx1 = _bsum(jnp.where(s0, fx1, 0.0))
    sy1 = _bsum(jnp.where(s0, fy1, 0.0))
    sx2 = _bsum(jnp.where(s0, fx2, 0.0))
    sy2 = _bsum(jnp.where(s0, fy2, 0.0))

    def body(i, carry):
        s, best, bx1, by1, bx2, by2, bsc, bcl = carry
        neg = best == _NEG
        eq = s == best
        gx1 = _bsum(jnp.where(eq, fx1, 0.0))
        gy1 = _bsum(jnp.where(eq, fy1, 0.0))
        gx2 = _bsum(jnp.where(eq, fx2, 0.0))
        gy2 = _bsum(jnp.where(eq, fy2, 0.0))
        gid = _bsum(jnp.where(eq, fid, 0.0))
        gx1 = jnp.where(neg, sx1, gx1)
        gy1 = jnp.where(neg, sy1, gy1)
        gx2 = jnp.where(neg, sx2, gx2)
        gy2 = jnp.where(neg, sy2, gy2)
        gid = jnp.where(neg, 0.0, gid)
        cls = jnp.mod(gid.astype(jnp.int32), _K)
        co = cls.astype(jnp.float32) * _OFFSET
        a1 = (gx2 - gx1) * (gy2 - gy1)
        xx1 = jnp.maximum(gx1, fx1)
        yy1 = jnp.maximum(gy1, fy1)
        xx2 = jnp.minimum(gx2, fx2)
        yy2 = jnp.minimum(gy2, fy2)
        inter = jnp.maximum(xx2 - xx1, 0.0) * jnp.maximum(yy2 - yy1, 0.0)
        iou = inter / (a1 + area - inter + 1e-9)
        s = jnp.where(iou > _NMS_THRESH, _NEG, s)
        nbest = _bmax(s)
        pick = lane == i
        return (s, nbest,
                jnp.where(pick, gx1 - co, bx1), jnp.where(pick, gy1 - co, by1),
                jnp.where(pick, gx2 - co, bx2), jnp.where(pick, gy2 - co, by2),
                jnp.where(pick, best, bsc),
                jnp.where(pick, cls.astype(jnp.float32), bcl))

    zf = jnp.zeros((1, 128), jnp.float32)
    carry = (s0v, _bmax(s0v), zf, zf, zf, zf, zf, zf)
    out = lax.fori_loop(0, _DETS, body, carry)
    _, _, bx1, by1, bx2, by2, bsc, bcl = out
    out_ref[...] = jnp.concatenate([bx1, by1, bx2, by2, bsc, bcl], axis=0)


def _nms(tab):
    return pl.pallas_call(
        _nms_body,
        out_shape=jax.ShapeDtypeStruct((6, 128), jnp.float32),
    )(tab)


def kernel(proposals, class_logits, box_deltas):
    d = box_deltas.reshape(_N, _K, 4)
    planes = _dense(
        proposals, class_logits, d[:, :, 0], d[:, :, 1], d[:, :, 2], d[:, :, 3])
    ctab = _compact(planes.reshape(-1))
    o = _nms(ctab.reshape(6, _ROWS, 128))
    det_boxes = jnp.stack(
        [o[0, :_DETS], o[1, :_DETS], o[2, :_DETS], o[3, :_DETS]], axis=1)
    return det_boxes, o[4, :_DETS], o[5, :_DETS].astype(jnp.int32)


# speedup vs baseline: 2.0560x; 2.0560x over previous
import math

import jax
import jax.numpy as jnp
from jax import lax
from jax.experimental import pallas as pl
from jax.experimental.pallas import tpu as pltpu
from jax.experimental.pallas import tpu_sc as plsc

_N = 1000
_NP = 1024
_K = 80
_SCORE_THRESH = 0.05
_NMS_THRESH = 0.5
_DETS = 100
_SCALE_CLAMP = math.log(1000.0 / 16.0)
_NEG = -1e9
_OFFSET = 4096.0

_NC = 2
_NS = 16
_NW = _NC * _NS
_FLAT = _NP * _K
_CHUNK = _FLAT // _NW
_UNROLL = 4
_VECS = _CHUNK // (16 * _UNROLL)
_CAP = 128
_TOT = _NW * _CAP
_ROWS = _TOT // 128


def _dense_body(prop_ref, logits_ref, dx_ref, dy_ref, dw_ref, dh_ref, out_ref):
    logits = logits_ref[...]
    m = jnp.max(logits, axis=1, keepdims=True)
    e = jnp.exp(logits - m)
    probs = e / jnp.sum(e, axis=1, keepdims=True)
    sc = probs[:, :_K]
    out_ref[0, 0:_N, :] = jnp.where(sc > _SCORE_THRESH, sc, _NEG)
    out_ref[0, _N:_NP, :] = jnp.full((_NP - _N, _K), _NEG, jnp.float32)

    p = prop_ref[...]
    w = p[:, 2:3] - p[:, 0:1]
    h = p[:, 3:4] - p[:, 1:2]
    cx = p[:, 0:1] + 0.5 * w
    cy = p[:, 1:2] + 0.5 * h
    dx = dx_ref[...] / 10.0
    dy = dy_ref[...] / 10.0
    dw = jnp.minimum(dw_ref[...] / 5.0, _SCALE_CLAMP)
    dh = jnp.minimum(dh_ref[...] / 5.0, _SCALE_CLAMP)
    pcx = dx * w + cx
    pcy = dy * h + cy
    pw = jnp.exp(dw) * w
    ph = jnp.exp(dh) * h
    off = lax.broadcasted_iota(jnp.int32, (_N, _K), 1).astype(jnp.float32) * _OFFSET
    zpad = jnp.zeros((_NP - _N, _K), jnp.float32)
    out_ref[1, 0:_N, :] = (pcx - 0.5 * pw) + off
    out_ref[1, _N:_NP, :] = zpad
    out_ref[2, 0:_N, :] = (pcy - 0.5 * ph) + off
    out_ref[2, _N:_NP, :] = zpad
    out_ref[3, 0:_N, :] = (pcx + 0.5 * pw) + off
    out_ref[3, _N:_NP, :] = zpad
    out_ref[4, 0:_N, :] = (pcy + 0.5 * ph) + off
    out_ref[4, _N:_NP, :] = zpad


def _dense(proposals, class_logits, dx, dy, dw, dh):
    return pl.pallas_call(
        _dense_body,
        out_shape=jax.ShapeDtypeStruct((5, _NP, _K), jnp.float32),
    )(proposals, class_logits, dx, dy, dw, dh)


def _compact_body(planes_hbm, out_hbm,
                  s_v, x1_v, y1_v, x2_v, y2_v, idx_v,
                  os_v, ox1_v, oy1_v, ox2_v, oy2_v, oid_v):
    cid = lax.axis_index("c")
    sid = lax.axis_index("s")
    wid = sid * _NC + cid
    base = wid * _CHUNK

    pltpu.sync_copy(planes_hbm.at[pl.ds(0 * _FLAT + base, _CHUNK)], s_v)
    pltpu.sync_copy(planes_hbm.at[pl.ds(1 * _FLAT + base, _CHUNK)], x1_v)
    pltpu.sync_copy(planes_hbm.at[pl.ds(2 * _FLAT + base, _CHUNK)], y1_v)
    pltpu.sync_copy(planes_hbm.at[pl.ds(3 * _FLAT + base, _CHUNK)], x2_v)
    pltpu.sync_copy(planes_hbm.at[pl.ds(4 * _FLAT + base, _CHUNK)], y2_v)

    zero16 = jnp.zeros((16,), jnp.int32)
    for j in range(_CAP // 16):
        idx_v[pl.ds(j * 16, 16)] = zero16

    lane = lax.iota(jnp.int32, 16)
    start = jnp.zeros((16,), jnp.int32) + jnp.where(wid == 0, 1, 0)

    def scan_body(v, cnt):
        b = v * (16 * _UNROLL)
        for u in range(_UNROLL):
            sv = s_v[pl.ds(b + u * 16, 16)]
            m = sv > _SCORE_THRESH
            pos = plsc.cumsum(jnp.where(m, 1, 0))
            dst = cnt + pos - 1
            m2 = jnp.logical_and(m, dst < _CAP)
            plsc.store_scatter(idx_v, [dst], lane + (b + u * 16), mask=m2)
            cnt = cnt + plsc.all_reduce_population_count(m2)
        return cnt

    cnt = lax.fori_loop(0, _VECS, scan_body, start)

    w0 = wid == 0
    for j in range(_CAP // 16):
        iv = idx_v[pl.ds(j * 16, 16)]
        gx1 = plsc.load_gather(x1_v, [iv])
        gy1 = plsc.load_gather(y1_v, [iv])
        gx2 = plsc.load_gather(x2_v, [iv])
        gy2 = plsc.load_gather(y2_v, [iv])
        gs = plsc.load_gather(s_v, [iv])
        valid = (lane + j * 16) < cnt
        gs = jnp.where(valid, gs, _NEG)
        if j == 0:
            gs = jnp.where(jnp.logical_and(w0, lane == 0), _NEG, gs)
        sl = pl.ds(j * 16, 16)
        os_v[sl] = gs
        ox1_v[sl] = gx1
        oy1_v[sl] = gy1
        ox2_v[sl] = gx2
        oy2_v[sl] = gy2
        oid_v[sl] = (iv + base).astype(jnp.float32)

    pltpu.sync_copy(os_v, out_hbm.at[0, wid])
    pltpu.sync_copy(ox1_v, out_hbm.at[1, wid])
    pltpu.sync_copy(oy1_v, out_hbm.at[2, wid])
    pltpu.sync_copy(ox2_v, out_hbm.at[3, wid])
    pltpu.sync_copy(oy2_v, out_hbm.at[4, wid])
    pltpu.sync_copy(oid_v, out_hbm.at[5, wid])


def _compact(planes_flat):
    chunk = pltpu.VMEM((_CHUNK,), jnp.float32)
    seg_f = pltpu.VMEM((_CAP,), jnp.float32)
    seg_i = pltpu.VMEM((_CAP,), jnp.int32)
    mesh = plsc.VectorSubcoreMesh(
        core_axis_name="c", subcore_axis_name="s",
        num_cores=_NC, num_subcores=_NS)
    run = pl.kernel(
        _compact_body,
        out_type=jax.ShapeDtypeStruct((6, _NW, _CAP), jnp.float32),
        mesh=mesh,
        scratch_types=[chunk] * 5 + [seg_i, seg_f, seg_f, seg_f, seg_f, seg_f, seg_f],
        compiler_params=pltpu.CompilerParams(needs_layout_passes=False),
    )
    return run(planes_flat)


def _nms_body(tab_ref, out_ref):
    s0v = tab_ref[0]
    fx1 = tab_ref[1]
    fy1 = tab_ref[2]
    fx2 = tab_ref[3]
    fy2 = tab_ref[4]
    fid = tab_ref[5]
    area = (fx2 - fx1) * (fy2 - fy1)
    pos = (lax.broadcasted_iota(jnp.int32, (_ROWS, 128), 0) * 128
           + lax.broadcasted_iota(jnp.int32, (_ROWS, 128), 1))
    lane = lax.broadcasted_iota(jnp.int32, (1, 128), 1)
    s0 = pos == 0
    sx1 = jnp.sum(jnp.where(s0, fx1, 0.0))
    sy1 = jnp.sum(jnp.where(s0, fy1, 0.0))
    sx2 = jnp.sum(jnp.where(s0, fx2, 0.0))
    sy2 = jnp.sum(jnp.where(s0, fy2, 0.0))

    def body(i, carry):
        s, best, bx1, by1, bx2, by2, bsc, bcl = carry
        neg = best == _NEG
        eq = s == best
        gx1 = jnp.sum(jnp.where(eq, fx1, 0.0))
        gy1 = jnp.sum(jnp.where(eq, fy1, 0.0))
        gx2 = jnp.sum(jnp.where(eq, fx2, 0.0))
        gy2 = jnp.sum(jnp.where(eq, fy2, 0.0))
        gid = jnp.sum(jnp.where(eq, fid, 0.0))
        gx1 = jnp.where(neg, sx1, gx1)
        gy1 = jnp.where(neg, sy1, gy1)
        gx2 = jnp.where(neg, sx2, gx2)
        gy2 = jnp.where(neg, sy2, gy2)
        gid = jnp.where(neg, 0.0, gid)
        cls = jnp.mod(gid.astype(jnp.int32), _K)
        co = cls.astype(jnp.float32) * _OFFSET
        a1 = (gx2 - gx1) * (gy2 - gy1)
        xx1 = jnp.maximum(gx1, fx1)
        yy1 = jnp.maximum(gy1, fy1)
        xx2 = jnp.minimum(gx2, fx2)
        yy2 = jnp.minimum(gy2, fy2)
        inter = jnp.maximum(xx2 - xx1, 0.0) * jnp.maximum(yy2 - yy1, 0.0)
        iou = inter / (a1 + area - inter + 1e-9)
        s = jnp.where(iou > _NMS_THRESH, _NEG, s)
        nbest = jnp.max(s)
        pick = lane == i
        return (s, nbest,
                jnp.where(pick, gx1 - co, bx1), jnp.where(pick, gy1 - co, by1),
                jnp.where(pick, gx2 - co, bx2), jnp.where(pick, gy2 - co, by2),
                jnp.where(pick, best, bsc),
                jnp.where(pick, cls.astype(jnp.float32), bcl))

    zf = jnp.zeros((1, 128), jnp.float32)
    carry = (s0v, jnp.max(s0v), zf, zf, zf, zf, zf, zf)
    out = lax.fori_loop(0, _DETS, body, carry)
    _, _, bx1, by1, bx2, by2, bsc, bcl = out
    out_ref[...] = jnp.concatenate([bx1, by1, bx2, by2, bsc, bcl], axis=0)


def _nms(tab):
    return pl.pallas_call(
        _nms_body,
        out_shape=jax.ShapeDtypeStruct((6, 128), jnp.float32),
    )(tab)


def kernel(proposals, class_logits, box_deltas):
    d = box_deltas.reshape(_N, _K, 4)
    planes = _dense(
        proposals, class_logits, d[:, :, 0], d[:, :, 1], d[:, :, 2], d[:, :, 3])
    ctab = _compact(planes.reshape(-1))
    o = _nms(ctab.reshape(6, _ROWS, 128))
    det_boxes = jnp.stack(
        [o[0, :_DETS], o[1, :_DETS], o[2, :_DETS], o[3, :_DETS]], axis=1)
    return det_boxes, o[4, :_DETS], o[5, :_DETS].astype(jnp.int32)
